# Initial kernel scaffold; baseline (speedup 1.0000x reference)
import jax
import jax.numpy as jnp
from jax.experimental import pallas as pl


def kernel(x, edge_index, W1, a_src1, a_dst1, b1, W2, a_src2, a_dst2, b2, Wv, a_srcv, a_dstv, bv):
    raise NotImplementedError("write your pallas kernel here")


# speedup vs baseline: 18.7394x; 18.7394x over previous
import functools

import jax
import jax.numpy as jnp
from jax import lax
from jax.experimental import pallas as pl
from jax.experimental.pallas import tpu as pltpu
from jax.experimental.pallas import tpu_sc as plsc

N = 10000
E = 320000
D = 128
DH = 64

NC = 2
NS = 16
L = 16
NW = NC * NS
EPT = E // NW
CH = 80
NCHUNK = EPT // CH
NPAD = 10240
RPT = NPAD // NS
ZR = 128
BN = 1000
EPS = 1e-16
F32 = jnp.float32


def _lrelu(x):
    return jnp.maximum(x, 0.2 * x)


_B_X = pl.BlockSpec((BN, D), lambda i: (i, 0))
_B_W = pl.BlockSpec((D, D), lambda i: (0, 0))
_B_A = pl.BlockSpec((D, 1), lambda i: (0, 0))
_B_BIAS = pl.BlockSpec((1, D), lambda i: (0, 0))
_B_NUM = pl.BlockSpec((NC, BN, DH), lambda i: (0, i, 0))
_B_DEN = pl.BlockSpec((BN, NC), lambda i: (i, 0))
_B_H = pl.BlockSpec((BN, DH), lambda i: (i, 0))
_B_V = pl.BlockSpec((BN, 1), lambda i: (i, 0))
_B_C = pl.BlockSpec((1, L), lambda i: (0, 0))

_GRID = (N // BN,)

_PROJ_OUT = [
    jax.ShapeDtypeStruct((N, DH), F32),
    jax.ShapeDtypeStruct((N, DH), F32),
    jax.ShapeDtypeStruct((N, 1), F32),
    jax.ShapeDtypeStruct((N, 1), F32),
]
_PROJ_OUT_SPECS = [_B_H, _B_H, _B_V, _B_V]


def _proj_body(h, asr_ref, ads_ref, h0_ref, h1_ref, av_ref, bv_ref):
    h0_ref[...] = h[:, :DH]
    h1_ref[...] = h[:, DH:]
    av_ref[...] = jnp.dot(h, asr_ref[...], preferred_element_type=F32)
    bv_ref[...] = jnp.dot(h, ads_ref[...], preferred_element_type=F32)


def _tc_pre_body(x_ref, w_ref, asr_ref, ads_ref,
                 h0_ref, h1_ref, av_ref, bv_ref):
    h = jnp.dot(x_ref[...], w_ref[...], preferred_element_type=F32)
    _proj_body(h, asr_ref, ads_ref, h0_ref, h1_ref, av_ref, bv_ref)


def _tc_pre(x, w, asr, ads):
    return pl.pallas_call(
        _tc_pre_body, grid=_GRID,
        in_specs=[_B_X, _B_W, _B_A, _B_A],
        out_specs=_PROJ_OUT_SPECS,
        out_shape=_PROJ_OUT,
    )(x, w, asr, ads)


def _numsum(num0_ref, num1_ref):
    return jnp.concatenate(
        [num0_ref[0] + num0_ref[1], num1_ref[0] + num1_ref[1]], axis=1)


def _tc_mid_body(num0_ref, num1_ref, den_ref, b_ref, w_ref, asr_ref, ads_ref,
                 h0_ref, h1_ref, av_ref, bv_ref):
    ns = _numsum(num0_ref, num1_ref)
    dsum = den_ref[:, 0:1] + den_ref[:, 1:2] + EPS
    o = ns / dsum + b_ref[...]
    o = jnp.maximum(o, 0.0)
    h = jnp.dot(o, w_ref[...], preferred_element_type=F32)
    _proj_body(h, asr_ref, ads_ref, h0_ref, h1_ref, av_ref, bv_ref)


def _tc_mid(num0, num1, den, b, w, asr, ads):
    return pl.pallas_call(
        _tc_mid_body, grid=_GRID,
        in_specs=[_B_NUM, _B_NUM, _B_DEN, _B_BIAS, _B_W, _B_A, _B_A],
        out_specs=_PROJ_OUT_SPECS,
        out_shape=_PROJ_OUT,
    )(num0, num1, den, b, w, asr, ads)


def _tc_mid3_body(num0_ref, num1_ref, den_ref, b_ref, w_ref, asr_ref, ads_ref,
                  mean_ref, h0_ref, h1_ref, av_ref, bv_ref, rcp_ref):
    ns = _numsum(num0_ref, num1_ref)
    dsum = den_ref[:, 0:1] + den_ref[:, 1:2] + EPS
    rcp = 1.0 / dsum
    mean = ns * rcp + b_ref[...]
    mean_ref[...] = mean
    rcp_ref[...] = rcp
    h = jnp.dot(mean, w_ref[...], preferred_element_type=F32)
    _proj_body(h, asr_ref, ads_ref, h0_ref, h1_ref, av_ref, bv_ref)


def _tc_mid3(num0, num1, den, b, w, asr, ads):
    return pl.pallas_call(
        _tc_mid3_body, grid=_GRID,
        in_specs=[_B_NUM, _B_NUM, _B_DEN, _B_BIAS, _B_W, _B_A, _B_A],
        out_specs=[pl.BlockSpec((BN, D), lambda i: (i, 0))]
        + _PROJ_OUT_SPECS + [_B_V],
        out_shape=[jax.ShapeDtypeStruct((N, D), F32)] + _PROJ_OUT
        + [jax.ShapeDtypeStruct((N, 1), F32)],
    )(num0, num1, den, b, w, asr, ads)


def _tc_fin_body(num0_ref, num1_ref, den_ref, b_ref, h0_ref, h1_ref,
                 av_ref, bv_ref, c_ref, var_ref):
    s = av_ref[...] + bv_ref[...]
    es = jnp.exp(_lrelu(s) - c_ref[0:1, 0:1])
    h = jnp.concatenate([h0_ref[...], h1_ref[...]], axis=1)
    numt = _numsum(num0_ref, num1_ref) + es * h
    dent = den_ref[:, 0:1] + den_ref[:, 1:2] + es + EPS
    var_ref[...] = numt / dent + b_ref[...]


def _tc_fin(num0, num1, den, b, h0, h1, av, bv, c):
    return pl.pallas_call(
        _tc_fin_body, grid=_GRID,
        in_specs=[_B_NUM, _B_NUM, _B_DEN, _B_BIAS, _B_H, _B_H, _B_V, _B_V,
                  _B_C],
        out_specs=pl.BlockSpec((BN, D), lambda i: (i, 0)),
        out_shape=jax.ShapeDtypeStruct((N, D), F32),
    )(num0, num1, den, b, h0, h1, av, bv, c)


def _make_sc_edge(mode):
    mesh = plsc.VectorSubcoreMesh(core_axis_name="c", subcore_axis_name="s",
                                  num_cores=NC, num_subcores=NS)

    out_type = [
        jax.ShapeDtypeStruct((NC, NPAD, DH), F32),
        jax.ShapeDtypeStruct((NC, NPAD, DH), F32),
        jax.ShapeDtypeStruct((NC * NPAD,), F32),
    ]
    if mode == 1:
        out_type.append(jax.ShapeDtypeStruct((NW, NCHUNK, CH), F32))
    if mode == 2:
        out_type.append(jax.ShapeDtypeStruct((NW, NCHUNK, CH), F32))
        out_type.append(jax.ShapeDtypeStruct((1, L), F32))

    scratch = [
        pltpu.VMEM((NCHUNK, CH), jnp.int32),
        pltpu.VMEM((NCHUNK, CH), jnp.int32),
        pltpu.VMEM((N,), F32),
        pltpu.VMEM((N,), F32),
        pltpu.VMEM((CH, DH), F32),
        pltpu.VMEM((NCHUNK, CH), F32),
        pltpu.VMEM((ZR, DH), F32),
        pltpu.VMEM((RPT,), F32),
        pltpu.VMEM((1, L), F32),
        pltpu.VMEM_SHARED((NPAD, DH), F32),
        pltpu.VMEM_SHARED((NPAD,), F32),
    ]
    if mode == 2:
        scratch += [
            pltpu.VMEM((CH,), F32),
        ]

    def body(*refs):
        if mode == 0:
            (h0_hbm, h1_hbm, asrc_hbm, adst_hbm, src_hbm, dst_hbm,
             num0_out, num1_out, den_out,
             src_l, dst_l, asrc_l, adst_l, rows, epf, zrow, zden, cw_l,
             accN, accD) = refs
        elif mode == 1:
            (h0_hbm, h1_hbm, asrc_hbm, adst_hbm, src_hbm, dst_hbm,
             num0_out, num1_out, den_out, ep_out,
             src_l, dst_l, asrc_l, adst_l, rows, epf, zrow, zden, cw_l,
             accN, accD) = refs
        else:
            (h0_hbm, h1_hbm, asrc_hbm, adst_hbm, src_hbm, dst_hbm,
             ep2_hbm, rcp_hbm,
             num0_out, num1_out, den_out, alpha_out, c_out,
             src_l, dst_l, asrc_l, adst_l, rows, epf, zrow, zden, cw_l,
             accN, accD, ep2c) = refs

        cid = lax.axis_index("c")
        sid = lax.axis_index("s")
        wid = cid * NS + sid

        zero = jnp.zeros((L,), F32)

        def zr_body(i, _):
            zrow[i // (DH // L), pl.ds((i % (DH // L)) * L, L)] = zero
            return 0

        def zd_body(i, _):
            zden[pl.ds(i * L, L)] = zero
            return 0
        lax.fori_loop(0, RPT // L, zd_body, 0)

        def zero_accN():
            lax.fori_loop(0, ZR * (DH // L), zr_body, 0)
            for k in range(RPT // ZR):
                pltpu.sync_copy(zrow, accN.at[pl.ds(sid * RPT + k * ZR, ZR)])

        def flush_accN(out):
            for k in range(RPT // ZR):
                sl = pl.ds(sid * RPT + k * ZR, ZR)
                pltpu.sync_copy(accN.at[sl], zrow)
                pltpu.sync_copy(zrow, out.at[cid, sl])

        zero_accN()
        pltpu.sync_copy(zden, accD.at[pl.ds(sid * RPT, RPT)])

        pltpu.sync_copy(src_hbm.at[wid], src_l)
        pltpu.sync_copy(dst_hbm.at[wid], dst_l)

        if mode == 2:
            pltpu.sync_copy(rcp_hbm, asrc_l)

            def apass(j, _):
                pltpu.sync_copy(ep2_hbm.at[wid, j], ep2c)
                for g in range(CH // L):
                    dv = dst_l[j, pl.ds(g * L, L)]
                    rv = plsc.load_gather(asrc_l, [dv])
                    epf[j, pl.ds(g * L, L)] = ep2c[pl.ds(g * L, L)] * rv
                return 0
            lax.fori_loop(0, NCHUNK, apass, 0)
            pltpu.sync_copy(epf, alpha_out.at[wid])

        pltpu.sync_copy(asrc_hbm, asrc_l)
        pltpu.sync_copy(adst_hbm, adst_l)

        plsc.subcore_barrier()

        ninf = jnp.full((L,), -3.0e38, F32)

        def mx_body(i, acc):
            ma, mb = acc
            return (jnp.maximum(ma, asrc_l[pl.ds(i * L, L)]),
                    jnp.maximum(mb, adst_l[pl.ds(i * L, L)]))
        ma, mb = lax.fori_loop(0, N // L, mx_body, (ninf, ninf))
        cs = jnp.max(ma) + jnp.max(mb)
        cval = jnp.full((L,), _lrelu(cs), F32)

        if mode == 2:
            cw_l[0, :] = cval

            @pl.when(wid == 0)
            def _():
                pltpu.sync_copy(cw_l, c_out)

        def make_chunk(h_hbm, first):
            def chunk(j, _):
                pltpu.sync_copy(h_hbm.at[src_l.at[j]], rows)
                if first:
                    for g in range(CH // L):
                        sv = src_l[j, pl.ds(g * L, L)]
                        dv = dst_l[j, pl.ds(g * L, L)]
                        s = (plsc.load_gather(asrc_l, [sv])
                             + plsc.load_gather(adst_l, [dv]))
                        epf[j, pl.ds(g * L, L)] = jnp.exp(_lrelu(s) - cval)
                    pltpu.sync_copy(epf.at[j], accD.at[dst_l.at[j]],
                                    add=True)

                def scale(r, _):
                    w = plsc.load_gather(
                        epf, [jnp.full((L,), j, jnp.int32),
                              jnp.full((L,), r, jnp.int32)])
                    for q in range(DH // L):
                        rows[r, pl.ds(q * L, L)] = (
                            rows[r, pl.ds(q * L, L)] * w)
                    return 0
                lax.fori_loop(0, CH, scale, 0)
                pltpu.sync_copy(rows, accN.at[dst_l.at[j]], add=True)
                return 0
            return chunk

        lax.fori_loop(0, NCHUNK, make_chunk(h0_hbm, True), 0)
        if mode == 1:
            pltpu.sync_copy(epf, ep_out.at[wid])
        plsc.subcore_barrier()
        flush_accN(num0_out)
        zero_accN()
        plsc.subcore_barrier()
        lax.fori_loop(0, NCHUNK, make_chunk(h1_hbm, False), 0)
        plsc.subcore_barrier()
        flush_accN(num1_out)

        pltpu.sync_copy(accD.at[pl.ds(sid * RPT, RPT)], zden)
        pltpu.sync_copy(zden,
                        den_out.at[pl.ds(cid * NPAD + sid * RPT, RPT)])

    return pl.kernel(body, out_type=out_type, mesh=mesh,
                     scratch_types=scratch,
                     compiler_params=pltpu.CompilerParams(
                         needs_layout_passes=False,
                         use_tc_tiling_on_sc=False))


_get_sc_edge = functools.lru_cache(maxsize=None)(_make_sc_edge)


def kernel(x, edge_index, W1, a_src1, a_dst1, b1,
           W2, a_src2, a_dst2, b2, Wv, a_srcv, a_dstv, bv):
    src = edge_index[0].reshape(NW, NCHUNK, CH)
    dst = edge_index[1].reshape(NW, NCHUNK, CH)

    def den2d(den_flat):
        return den_flat.reshape(NC, NPAD)[:, :N].T

    h1a, h1b, av1, bv1 = _tc_pre(x, W1, a_src1.reshape(D, 1),
                                 a_dst1.reshape(D, 1))
    n1a, n1b, den1 = _get_sc_edge(0)(h1a, h1b, av1.reshape(N),
                                     bv1.reshape(N), src, dst)

    h2a, h2b, av2, bv2 = _tc_mid(n1a, n1b, den2d(den1), b1.reshape(1, D),
                                 W2, a_src2.reshape(D, 1),
                                 a_dst2.reshape(D, 1))
    n2a, n2b, den2, ep2 = _get_sc_edge(1)(h2a, h2b, av2.reshape(N),
                                          bv2.reshape(N), src, dst)

    mean, h3a, h3b, av3, bv3, rcp2 = _tc_mid3(
        n2a, n2b, den2d(den2), b2.reshape(1, D), Wv,
        a_srcv.reshape(D, 1), a_dstv.reshape(D, 1))
    n3a, n3b, den3, alpha, c3 = _get_sc_edge(2)(h3a, h3b, av3.reshape(N),
                                                bv3.reshape(N), src, dst,
                                                ep2, rcp2.reshape(N))

    var = _tc_fin(n3a, n3b, den2d(den3), bv.reshape(1, D), h3a, h3b,
                  av3, bv3, c3)
    return (mean, var, alpha.reshape(E))


# speedup vs baseline: 29.8566x; 1.5933x over previous
import functools

import jax
import jax.numpy as jnp
from jax import lax
from jax.experimental import pallas as pl
from jax.experimental.pallas import tpu as pltpu
from jax.experimental.pallas import tpu_sc as plsc

N = 10000
E = 320000
D = 128
DH = 64

NC = 2
NS = 16
L = 16
NW = NC * NS
EPT = E // NW
CH = 80
NCHUNK = EPT // CH
NB = 5
NG = NCHUNK // NB
NPAD = 10240
RPT = NPAD // NS
ZR = 128
BN = 1000
EPS = 1e-16
F32 = jnp.float32


def _lrelu(x):
    return jnp.maximum(x, 0.2 * x)


_B_X = pl.BlockSpec((BN, D), lambda i: (i, 0))
_B_W = pl.BlockSpec((D, D), lambda i: (0, 0))
_B_A = pl.BlockSpec((D, 1), lambda i: (0, 0))
_B_BIAS = pl.BlockSpec((1, D), lambda i: (0, 0))
_B_NUM = pl.BlockSpec((NC, BN, DH), lambda i: (0, i, 0))
_B_DEN = pl.BlockSpec((BN, NC), lambda i: (i, 0))
_B_H = pl.BlockSpec((BN, DH), lambda i: (i, 0))
_B_V = pl.BlockSpec((BN, 1), lambda i: (i, 0))
_B_C = pl.BlockSpec((1, L), lambda i: (0, 0))

_GRID = (N // BN,)

_PROJ_OUT = [
    jax.ShapeDtypeStruct((N, DH), F32),
    jax.ShapeDtypeStruct((N, DH), F32),
    jax.ShapeDtypeStruct((N, 1), F32),
    jax.ShapeDtypeStruct((N, 1), F32),
]
_PROJ_OUT_SPECS = [_B_H, _B_H, _B_V, _B_V]


def _proj_body(h, asr_ref, ads_ref, h0_ref, h1_ref, av_ref, bv_ref):
    h0_ref[...] = h[:, :DH]
    h1_ref[...] = h[:, DH:]
    av_ref[...] = jnp.dot(h, asr_ref[...], preferred_element_type=F32)
    bv_ref[...] = jnp.dot(h, ads_ref[...], preferred_element_type=F32)


def _tc_pre_body(x_ref, w_ref, asr_ref, ads_ref,
                 h0_ref, h1_ref, av_ref, bv_ref):
    h = jnp.dot(x_ref[...], w_ref[...], preferred_element_type=F32)
    _proj_body(h, asr_ref, ads_ref, h0_ref, h1_ref, av_ref, bv_ref)


def _tc_pre(x, w, asr, ads):
    return pl.pallas_call(
        _tc_pre_body, grid=_GRID,
        in_specs=[_B_X, _B_W, _B_A, _B_A],
        out_specs=_PROJ_OUT_SPECS,
        out_shape=_PROJ_OUT,
    )(x, w, asr, ads)


def _numsum(num0_ref, num1_ref):
    return jnp.concatenate(
        [num0_ref[0] + num0_ref[1], num1_ref[0] + num1_ref[1]], axis=1)


def _tc_mid_body(num0_ref, num1_ref, den_ref, b_ref, w_ref, asr_ref, ads_ref,
                 h0_ref, h1_ref, av_ref, bv_ref):
    ns = _numsum(num0_ref, num1_ref)
    dsum = den_ref[:, 0:1] + den_ref[:, 1:2] + EPS
    o = ns / dsum + b_ref[...]
    o = jnp.maximum(o, 0.0)
    h = jnp.dot(o, w_ref[...], preferred_element_type=F32)
    _proj_body(h, asr_ref, ads_ref, h0_ref, h1_ref, av_ref, bv_ref)


def _tc_mid(num0, num1, den, b, w, asr, ads):
    return pl.pallas_call(
        _tc_mid_body, grid=_GRID,
        in_specs=[_B_NUM, _B_NUM, _B_DEN, _B_BIAS, _B_W, _B_A, _B_A],
        out_specs=_PROJ_OUT_SPECS,
        out_shape=_PROJ_OUT,
    )(num0, num1, den, b, w, asr, ads)


def _tc_mid3_body(num0_ref, num1_ref, den_ref, b_ref, w_ref, asr_ref, ads_ref,
                  mean_ref, h0_ref, h1_ref, av_ref, bv_ref, rcp_ref):
    ns = _numsum(num0_ref, num1_ref)
    dsum = den_ref[:, 0:1] + den_ref[:, 1:2] + EPS
    rcp = 1.0 / dsum
    mean = ns * rcp + b_ref[...]
    mean_ref[...] = mean
    rcp_ref[...] = rcp
    h = jnp.dot(mean, w_ref[...], preferred_element_type=F32)
    _proj_body(h, asr_ref, ads_ref, h0_ref, h1_ref, av_ref, bv_ref)


def _tc_mid3(num0, num1, den, b, w, asr, ads):
    return pl.pallas_call(
        _tc_mid3_body, grid=_GRID,
        in_specs=[_B_NUM, _B_NUM, _B_DEN, _B_BIAS, _B_W, _B_A, _B_A],
        out_specs=[pl.BlockSpec((BN, D), lambda i: (i, 0))]
        + _PROJ_OUT_SPECS + [_B_V],
        out_shape=[jax.ShapeDtypeStruct((N, D), F32)] + _PROJ_OUT
        + [jax.ShapeDtypeStruct((N, 1), F32)],
    )(num0, num1, den, b, w, asr, ads)


def _tc_fin_body(num0_ref, num1_ref, den_ref, b_ref, h0_ref, h1_ref,
                 av_ref, bv_ref, c_ref, var_ref):
    s = av_ref[...] + bv_ref[...]
    es = jnp.exp(_lrelu(s) - c_ref[0:1, 0:1])
    h = jnp.concatenate([h0_ref[...], h1_ref[...]], axis=1)
    numt = _numsum(num0_ref, num1_ref) + es * h
    dent = den_ref[:, 0:1] + den_ref[:, 1:2] + es + EPS
    var_ref[...] = numt / dent + b_ref[...]


def _tc_fin(num0, num1, den, b, h0, h1, av, bv, c):
    return pl.pallas_call(
        _tc_fin_body, grid=_GRID,
        in_specs=[_B_NUM, _B_NUM, _B_DEN, _B_BIAS, _B_H, _B_H, _B_V, _B_V,
                  _B_C],
        out_specs=pl.BlockSpec((BN, D), lambda i: (i, 0)),
        out_shape=jax.ShapeDtypeStruct((N, D), F32),
    )(num0, num1, den, b, h0, h1, av, bv, c)


def _make_sc_edge(mode):
    mesh = plsc.VectorSubcoreMesh(core_axis_name="c", subcore_axis_name="s",
                                  num_cores=NC, num_subcores=NS)

    out_type = [
        jax.ShapeDtypeStruct((NC, NPAD, DH), F32),
        jax.ShapeDtypeStruct((NC, NPAD, DH), F32),
        jax.ShapeDtypeStruct((NC * NPAD,), F32),
    ]
    if mode == 1:
        out_type.append(jax.ShapeDtypeStruct((NW, NCHUNK, CH), F32))
    if mode == 2:
        out_type.append(jax.ShapeDtypeStruct((NW, NCHUNK, CH), F32))
        out_type.append(jax.ShapeDtypeStruct((1, L), F32))

    scratch = [
        pltpu.VMEM((NCHUNK, CH), jnp.int32),
        pltpu.VMEM((NCHUNK, CH), jnp.int32),
        pltpu.VMEM((N,), F32),
        pltpu.VMEM((N,), F32),
        pltpu.VMEM((NB, CH, DH), F32),
        pltpu.VMEM((NCHUNK, CH), F32),
        pltpu.VMEM((ZR, DH), F32),
        pltpu.VMEM((RPT,), F32),
        pltpu.VMEM((1, L), F32),
        pltpu.VMEM_SHARED((NPAD, DH), F32),
        pltpu.VMEM_SHARED((NPAD,), F32),
        pltpu.SemaphoreType.DMA,
        pltpu.SemaphoreType.DMA,
        pltpu.SemaphoreType.DMA,
        pltpu.SemaphoreType.DMA,
        pltpu.SemaphoreType.DMA,
        pltpu.SemaphoreType.DMA,
        pltpu.SemaphoreType.DMA,
    ]

    def body(*refs):
        if mode == 0:
            (h0_hbm, h1_hbm, asrc_hbm, adst_hbm, src_hbm, dst_hbm,
             num0_out, num1_out, den_out,
             src_l, dst_l, asrc_l, adst_l, rows, epf, zrow, zden, cw_l,
             accN, accD, g0, g1, g2, g3, g4, ssem, dsem) = refs
        elif mode == 1:
            (h0_hbm, h1_hbm, asrc_hbm, adst_hbm, src_hbm, dst_hbm,
             num0_out, num1_out, den_out, ep_out,
             src_l, dst_l, asrc_l, adst_l, rows, epf, zrow, zden, cw_l,
             accN, accD, g0, g1, g2, g3, g4, ssem, dsem) = refs
        else:
            (h0_hbm, h1_hbm, asrc_hbm, adst_hbm, src_hbm, dst_hbm,
             ep2_hbm, rcp_hbm,
             num0_out, num1_out, den_out, alpha_out, c_out,
             src_l, dst_l, asrc_l, adst_l, rows, epf, zrow, zden, cw_l,
             accN, accD, g0, g1, g2, g3, g4, ssem, dsem) = refs
        gsems = (g0, g1, g2, g3, g4)

        cid = lax.axis_index("c")
        sid = lax.axis_index("s")
        wid = cid * NS + sid

        zero = jnp.zeros((L,), F32)

        def zr_body(i, _):
            zrow[i // (DH // L), pl.ds((i % (DH // L)) * L, L)] = zero
            return 0

        def zd_body(i, _):
            zden[pl.ds(i * L, L)] = zero
            return 0
        lax.fori_loop(0, RPT // L, zd_body, 0)

        def zero_accN():
            lax.fori_loop(0, ZR * (DH // L), zr_body, 0)
            for k in range(RPT // ZR):
                pltpu.sync_copy(zrow, accN.at[pl.ds(sid * RPT + k * ZR, ZR)])

        def flush_accN(out):
            for k in range(RPT // ZR):
                sl = pl.ds(sid * RPT + k * ZR, ZR)
                pltpu.sync_copy(accN.at[sl], zrow)
                pltpu.sync_copy(zrow, out.at[cid, sl])

        zero_accN()
        pltpu.sync_copy(zden, accD.at[pl.ds(sid * RPT, RPT)])

        pltpu.sync_copy(src_hbm.at[wid], src_l)
        pltpu.sync_copy(dst_hbm.at[wid], dst_l)

        if mode == 2:
            pltpu.sync_copy(rcp_hbm, asrc_l)
            pltpu.sync_copy(ep2_hbm.at[wid], epf)

            def apass(j, _):
                for g in range(CH // L):
                    dv = dst_l[j, pl.ds(g * L, L)]
                    rv = plsc.load_gather(asrc_l, [dv])
                    epf[j, pl.ds(g * L, L)] = epf[j, pl.ds(g * L, L)] * rv
                return 0
            lax.fori_loop(0, NCHUNK, apass, 0)
            pltpu.sync_copy(epf, alpha_out.at[wid])

        pltpu.sync_copy(asrc_hbm, asrc_l)
        pltpu.sync_copy(adst_hbm, adst_l)

        plsc.subcore_barrier()

        ninf = jnp.full((L,), -3.0e38, F32)

        def mx_body(i, acc):
            ma, mb = acc
            return (jnp.maximum(ma, asrc_l[pl.ds(i * L, L)]),
                    jnp.maximum(mb, adst_l[pl.ds(i * L, L)]))
        ma, mb = lax.fori_loop(0, N // L, mx_body, (ninf, ninf))
        cs = jnp.max(ma) + jnp.max(mb)
        cval = jnp.full((L,), _lrelu(cs), F32)

        if mode == 2:
            cw_l[0, :] = cval

            @pl.when(wid == 0)
            def _():
                pltpu.sync_copy(cw_l, c_out)

        def make_group(h_hbm, first):
            def group(gidx, _):
                j0 = gidx * NB
                for b in range(NB):
                    pltpu.async_copy(h_hbm.at[src_l.at[j0 + b]], rows.at[b],
                                     gsems[b])
                for b in range(NB):
                    j = j0 + b
                    if first:
                        for g in range(CH // L):
                            sv = src_l[j, pl.ds(g * L, L)]
                            dv = dst_l[j, pl.ds(g * L, L)]
                            s = (plsc.load_gather(asrc_l, [sv])
                                 + plsc.load_gather(adst_l, [dv]))
                            epf[j, pl.ds(g * L, L)] = jnp.exp(_lrelu(s)
                                                              - cval)
                        pltpu.async_copy(epf.at[j], accD.at[dst_l.at[j]],
                                         dsem, add=True)
                    pltpu.make_async_copy(h_hbm.at[src_l.at[j]], rows.at[b],
                                          gsems[b]).wait()

                    def scale(r, _):
                        w = plsc.load_gather(
                            epf, [jnp.full((L,), j, jnp.int32),
                                  jnp.full((L,), r, jnp.int32)])
                        for q in range(DH // L):
                            rows[b, r, pl.ds(q * L, L)] = (
                                rows[b, r, pl.ds(q * L, L)] * w)
                        return 0
                    lax.fori_loop(0, CH, scale, 0)
                    pltpu.async_copy(rows.at[b], accN.at[dst_l.at[j]], ssem,
                                     add=True)
                for b in range(NB):
                    j = j0 + b
                    pltpu.make_async_copy(rows.at[b], accN.at[dst_l.at[j]],
                                          ssem).wait()
                    if first:
                        pltpu.make_async_copy(epf.at[j],
                                              accD.at[dst_l.at[j]],
                                              dsem).wait()
                return 0
            return group

        lax.fori_loop(0, NG, make_group(h0_hbm, True), 0)
        if mode == 1:
            pltpu.sync_copy(epf, ep_out.at[wid])
        plsc.subcore_barrier()
        flush_accN(num0_out)
        zero_accN()
        plsc.subcore_barrier()
        lax.fori_loop(0, NG, make_group(h1_hbm, False), 0)
        plsc.subcore_barrier()
        flush_accN(num1_out)

        pltpu.sync_copy(accD.at[pl.ds(sid * RPT, RPT)], zden)
        pltpu.sync_copy(zden,
                        den_out.at[pl.ds(cid * NPAD + sid * RPT, RPT)])

    return pl.kernel(body, out_type=out_type, mesh=mesh,
                     scratch_types=scratch,
                     compiler_params=pltpu.CompilerParams(
                         needs_layout_passes=False,
                         use_tc_tiling_on_sc=False))


_get_sc_edge = functools.lru_cache(maxsize=None)(_make_sc_edge)


def kernel(x, edge_index, W1, a_src1, a_dst1, b1,
           W2, a_src2, a_dst2, b2, Wv, a_srcv, a_dstv, bv):
    src = edge_index[0].reshape(NW, NCHUNK, CH)
    dst = edge_index[1].reshape(NW, NCHUNK, CH)

    def den2d(den_flat):
        return den_flat.reshape(NC, NPAD)[:, :N].T

    h1a, h1b, av1, bv1 = _tc_pre(x, W1, a_src1.reshape(D, 1),
                                 a_dst1.reshape(D, 1))
    n1a, n1b, den1 = _get_sc_edge(0)(h1a, h1b, av1.reshape(N),
                                     bv1.reshape(N), src, dst)

    h2a, h2b, av2, bv2 = _tc_mid(n1a, n1b, den2d(den1), b1.reshape(1, D),
                                 W2, a_src2.reshape(D, 1),
                                 a_dst2.reshape(D, 1))
    n2a, n2b, den2, ep2 = _get_sc_edge(1)(h2a, h2b, av2.reshape(N),
                                          bv2.reshape(N), src, dst)

    mean, h3a, h3b, av3, bv3, rcp2 = _tc_mid3(
        n2a, n2b, den2d(den2), b2.reshape(1, D), Wv,
        a_srcv.reshape(D, 1), a_dstv.reshape(D, 1))
    n3a, n3b, den3, alpha, c3 = _get_sc_edge(2)(h3a, h3b, av3.reshape(N),
                                                bv3.reshape(N), src, dst,
                                                ep2, rcp2.reshape(N))

    var = _tc_fin(n3a, n3b, den2d(den3), bv.reshape(1, D), h3a, h3b,
                  av3, bv3, c3)
    return (mean, var, alpha.reshape(E))


# speedup vs baseline: 38.1575x; 1.2780x over previous
import functools

import jax
import jax.numpy as jnp
from jax import lax
from jax.experimental import pallas as pl
from jax.experimental.pallas import tpu as pltpu
from jax.experimental.pallas import tpu_sc as plsc

N = 10000
E = 320000
D = 128
DH = 64

NC = 2
NS = 16
L = 16
NW = NC * NS
EPT = E // NW
CH = 80
NCHUNK = EPT // CH
NB = 5
NG = NCHUNK // NB
NPAD = 10240
RPT = NPAD // NS
ZR = 128
BN = 1000
EPS = 1e-16
F32 = jnp.float32


def _lrelu(x):
    return jnp.maximum(x, 0.2 * x)


_B_X = pl.BlockSpec((BN, D), lambda i: (i, 0))
_B_W = pl.BlockSpec((D, D), lambda i: (0, 0))
_B_A = pl.BlockSpec((D, 1), lambda i: (0, 0))
_B_BIAS = pl.BlockSpec((1, D), lambda i: (0, 0))
_B_NUM = pl.BlockSpec((NC, BN, DH), lambda i: (0, i, 0))
_B_DEN = pl.BlockSpec((BN, NC), lambda i: (i, 0))
_B_H = pl.BlockSpec((BN, DH), lambda i: (i, 0))
_B_V = pl.BlockSpec((BN, 1), lambda i: (i, 0))
_B_C = pl.BlockSpec((1, L), lambda i: (0, 0))

_GRID = (N // BN,)

_PROJ_OUT = [
    jax.ShapeDtypeStruct((N, DH), F32),
    jax.ShapeDtypeStruct((N, DH), F32),
    jax.ShapeDtypeStruct((N, 1), F32),
    jax.ShapeDtypeStruct((N, 1), F32),
]
_PROJ_OUT_SPECS = [_B_H, _B_H, _B_V, _B_V]


def _proj_body(h, asr_ref, ads_ref, h0_ref, h1_ref, av_ref, bv_ref):
    h0_ref[...] = h[:, :DH]
    h1_ref[...] = h[:, DH:]
    av_ref[...] = jnp.dot(h, asr_ref[...], preferred_element_type=F32)
    bv_ref[...] = jnp.dot(h, ads_ref[...], preferred_element_type=F32)


def _tc_pre_body(x_ref, w_ref, asr_ref, ads_ref,
                 h0_ref, h1_ref, av_ref, bv_ref):
    h = jnp.dot(x_ref[...], w_ref[...], preferred_element_type=F32)
    _proj_body(h, asr_ref, ads_ref, h0_ref, h1_ref, av_ref, bv_ref)


def _tc_pre(x, w, asr, ads):
    return pl.pallas_call(
        _tc_pre_body, grid=_GRID,
        in_specs=[_B_X, _B_W, _B_A, _B_A],
        out_specs=_PROJ_OUT_SPECS,
        out_shape=_PROJ_OUT,
    )(x, w, asr, ads)


def _numsum(num0_ref, num1_ref):
    return jnp.concatenate(
        [num0_ref[0] + num0_ref[1], num1_ref[0] + num1_ref[1]], axis=1)


def _tc_mid_body(num0_ref, num1_ref, den_ref, b_ref, w_ref, asr_ref, ads_ref,
                 h0_ref, h1_ref, av_ref, bv_ref):
    ns = _numsum(num0_ref, num1_ref)
    dsum = den_ref[:, 0:1] + den_ref[:, 1:2] + EPS
    o = ns / dsum + b_ref[...]
    o = jnp.maximum(o, 0.0)
    h = jnp.dot(o, w_ref[...], preferred_element_type=F32)
    _proj_body(h, asr_ref, ads_ref, h0_ref, h1_ref, av_ref, bv_ref)


def _tc_mid(num0, num1, den, b, w, asr, ads):
    return pl.pallas_call(
        _tc_mid_body, grid=_GRID,
        in_specs=[_B_NUM, _B_NUM, _B_DEN, _B_BIAS, _B_W, _B_A, _B_A],
        out_specs=_PROJ_OUT_SPECS,
        out_shape=_PROJ_OUT,
    )(num0, num1, den, b, w, asr, ads)


def _tc_mid3_body(num0_ref, num1_ref, den_ref, b_ref, w_ref, asr_ref, ads_ref,
                  mean_ref, h0_ref, h1_ref, av_ref, bv_ref, rcp_ref):
    ns = _numsum(num0_ref, num1_ref)
    dsum = den_ref[:, 0:1] + den_ref[:, 1:2] + EPS
    rcp = 1.0 / dsum
    mean = ns * rcp + b_ref[...]
    mean_ref[...] = mean
    rcp_ref[...] = rcp
    h = jnp.dot(mean, w_ref[...], preferred_element_type=F32)
    _proj_body(h, asr_ref, ads_ref, h0_ref, h1_ref, av_ref, bv_ref)


def _tc_mid3(num0, num1, den, b, w, asr, ads):
    return pl.pallas_call(
        _tc_mid3_body, grid=_GRID,
        in_specs=[_B_NUM, _B_NUM, _B_DEN, _B_BIAS, _B_W, _B_A, _B_A],
        out_specs=[pl.BlockSpec((BN, D), lambda i: (i, 0))]
        + _PROJ_OUT_SPECS + [_B_V],
        out_shape=[jax.ShapeDtypeStruct((N, D), F32)] + _PROJ_OUT
        + [jax.ShapeDtypeStruct((N, 1), F32)],
    )(num0, num1, den, b, w, asr, ads)


def _tc_fin_body(num0_ref, num1_ref, den_ref, b_ref, h0_ref, h1_ref,
                 av_ref, bv_ref, c_ref, var_ref):
    s = av_ref[...] + bv_ref[...]
    es = jnp.exp(_lrelu(s) - c_ref[0:1, 0:1])
    h = jnp.concatenate([h0_ref[...], h1_ref[...]], axis=1)
    numt = _numsum(num0_ref, num1_ref) + es * h
    dent = den_ref[:, 0:1] + den_ref[:, 1:2] + es + EPS
    var_ref[...] = numt / dent + b_ref[...]


def _tc_fin(num0, num1, den, b, h0, h1, av, bv, c):
    return pl.pallas_call(
        _tc_fin_body, grid=_GRID,
        in_specs=[_B_NUM, _B_NUM, _B_DEN, _B_BIAS, _B_H, _B_H, _B_V, _B_V,
                  _B_C],
        out_specs=pl.BlockSpec((BN, D), lambda i: (i, 0)),
        out_shape=jax.ShapeDtypeStruct((N, D), F32),
    )(num0, num1, den, b, h0, h1, av, bv, c)


def _make_sc_edge(mode):
    mesh = plsc.VectorSubcoreMesh(core_axis_name="c", subcore_axis_name="s",
                                  num_cores=NC, num_subcores=NS)

    out_type = [
        jax.ShapeDtypeStruct((NC, NPAD, DH), F32),
        jax.ShapeDtypeStruct((NC, NPAD, DH), F32),
        jax.ShapeDtypeStruct((NC * NPAD,), F32),
    ]
    if mode == 1:
        out_type.append(jax.ShapeDtypeStruct((NW, NCHUNK, CH), F32))
    if mode == 2:
        out_type.append(jax.ShapeDtypeStruct((NW, NCHUNK, CH), F32))
        out_type.append(jax.ShapeDtypeStruct((1, L), F32))

    scratch = [
        pltpu.VMEM((NCHUNK, CH), jnp.int32),
        pltpu.VMEM((NCHUNK, CH), jnp.int32),
        pltpu.VMEM((N,), F32),
        pltpu.VMEM((N,), F32),
        pltpu.VMEM((NB, CH, DH), F32),
        pltpu.VMEM((NCHUNK, CH), F32),
        pltpu.VMEM((ZR, DH), F32),
        pltpu.VMEM((RPT,), F32),
        pltpu.VMEM((1, L), F32),
        pltpu.VMEM_SHARED((NPAD, DH), F32),
        pltpu.VMEM_SHARED((NPAD,), F32),
    ] + [pltpu.SemaphoreType.DMA] * (2 * NB + 1)

    def body(*refs):
        if mode == 0:
            (h0_hbm, h1_hbm, asrc_hbm, adst_hbm, src_hbm, dst_hbm,
             num0_out, num1_out, den_out,
             src_l, dst_l, asrc_l, adst_l, rows, epf, zrow, zden, cw_l,
             accN, accD, *sems) = refs
        elif mode == 1:
            (h0_hbm, h1_hbm, asrc_hbm, adst_hbm, src_hbm, dst_hbm,
             num0_out, num1_out, den_out, ep_out,
             src_l, dst_l, asrc_l, adst_l, rows, epf, zrow, zden, cw_l,
             accN, accD, *sems) = refs
        else:
            (h0_hbm, h1_hbm, asrc_hbm, adst_hbm, src_hbm, dst_hbm,
             ep2_hbm, rcp_hbm,
             num0_out, num1_out, den_out, alpha_out, c_out,
             src_l, dst_l, asrc_l, adst_l, rows, epf, zrow, zden, cw_l,
             accN, accD, *sems) = refs
        gsems = sems[:NB]
        ssems = sems[NB:2 * NB]
        dsem = sems[2 * NB]

        cid = lax.axis_index("c")
        sid = lax.axis_index("s")
        wid = cid * NS + sid

        zero = jnp.zeros((L,), F32)

        def zr_body(i, _):
            zrow[i // (DH // L), pl.ds((i % (DH // L)) * L, L)] = zero
            return 0

        def zd_body(i, _):
            zden[pl.ds(i * L, L)] = zero
            return 0
        lax.fori_loop(0, RPT // L, zd_body, 0)

        def zero_accN():
            lax.fori_loop(0, ZR * (DH // L), zr_body, 0)
            for k in range(RPT // ZR):
                pltpu.sync_copy(zrow, accN.at[pl.ds(sid * RPT + k * ZR, ZR)])

        def flush_accN(out):
            for k in range(RPT // ZR):
                sl = pl.ds(sid * RPT + k * ZR, ZR)
                pltpu.sync_copy(accN.at[sl], zrow)
                pltpu.sync_copy(zrow, out.at[cid, sl])

        zero_accN()
        pltpu.sync_copy(zden, accD.at[pl.ds(sid * RPT, RPT)])

        pltpu.sync_copy(src_hbm.at[wid], src_l)
        pltpu.sync_copy(dst_hbm.at[wid], dst_l)

        if mode == 2:
            pltpu.sync_copy(rcp_hbm, asrc_l)
            pltpu.sync_copy(ep2_hbm.at[wid], epf)

            def apass(j, _):
                for g in range(CH // L):
                    dv = dst_l[j, pl.ds(g * L, L)]
                    rv = plsc.load_gather(asrc_l, [dv])
                    epf[j, pl.ds(g * L, L)] = epf[j, pl.ds(g * L, L)] * rv
                return 0
            lax.fori_loop(0, NCHUNK, apass, 0)
            pltpu.sync_copy(epf, alpha_out.at[wid])

        pltpu.sync_copy(asrc_hbm, asrc_l)
        pltpu.sync_copy(adst_hbm, adst_l)

        plsc.subcore_barrier()

        ninf = jnp.full((L,), -3.0e38, F32)

        def mx_body(i, acc):
            ma, mb = acc
            return (jnp.maximum(ma, asrc_l[pl.ds(i * L, L)]),
                    jnp.maximum(mb, adst_l[pl.ds(i * L, L)]))
        ma, mb = lax.fori_loop(0, N // L, mx_body, (ninf, ninf))
        cs = jnp.max(ma) + jnp.max(mb)
        cval = jnp.full((L,), _lrelu(cs), F32)

        if mode == 2:
            cw_l[0, :] = cval

            @pl.when(wid == 0)
            def _():
                pltpu.sync_copy(cw_l, c_out)

        def do_pass(h_hbm, first):
            def fire_gather(j, b):
                pltpu.async_copy(h_hbm.at[src_l.at[j]], rows.at[b],
                                 gsems[b])

            def wait_gather(j, b):
                pltpu.make_async_copy(h_hbm.at[src_l.at[j]], rows.at[b],
                                      gsems[b]).wait()

            def fire_scat(j, b):
                pltpu.async_copy(rows.at[b], accN.at[dst_l.at[j]],
                                 ssems[b], add=True)

            def wait_scat(j, b):
                pltpu.make_async_copy(rows.at[b], accN.at[dst_l.at[j]],
                                      ssems[b]).wait()

            def wait_den(j):
                pltpu.make_async_copy(epf.at[j], accD.at[dst_l.at[j]],
                                      dsem).wait()

            def step(j, b, wait_prev_den):
                wait_gather(j, b)
                if first:
                    for g in range(CH // L):
                        sv = src_l[j, pl.ds(g * L, L)]
                        dv = dst_l[j, pl.ds(g * L, L)]
                        s = (plsc.load_gather(asrc_l, [sv])
                             + plsc.load_gather(adst_l, [dv]))
                        epf[j, pl.ds(g * L, L)] = jnp.exp(_lrelu(s) - cval)
                    if wait_prev_den:
                        wait_den(j - 1)
                    pltpu.async_copy(epf.at[j], accD.at[dst_l.at[j]],
                                     dsem, add=True)

                def scale(r, _):
                    w = plsc.load_gather(
                        epf, [jnp.full((L,), j, jnp.int32),
                              jnp.full((L,), r, jnp.int32)])
                    for q in range(DH // L):
                        rows[b, r, pl.ds(q * L, L)] = (
                            rows[b, r, pl.ds(q * L, L)] * w)
                    return 0
                lax.fori_loop(0, CH, scale, 0)
                fire_scat(j, b)

            for b in range(NB - 1):
                fire_gather(b, b)
            for b in range(NB):
                step(b, b, wait_prev_den=(first and b > 0))
                if b > 0:
                    wait_scat(b - 1, b - 1)
                fire_gather(b + NB - 1, (b - 1) % NB)

            def group(gidx, _):
                j0 = gidx * NB
                for b in range(NB):
                    j = j0 + b
                    step(j, b, wait_prev_den=first)
                    wait_scat(j - 1, (b + NB - 1) % NB)
                    fire_gather(j + NB - 1, (b + NB - 1) % NB)
                return 0
            lax.fori_loop(1, NG - 1, group, 0)

            j0 = NCHUNK - NB
            step(j0, 0, wait_prev_den=first)
            wait_scat(j0 - 1, NB - 1)
            fire_gather(NCHUNK - 1, NB - 1)
            for b in range(1, NB):
                step(j0 + b, b, wait_prev_den=first)
            for b in range(NB):
                wait_scat(j0 + b, b)
            if first:
                wait_den(NCHUNK - 1)

        do_pass(h0_hbm, True)
        if mode == 1:
            pltpu.sync_copy(epf, ep_out.at[wid])
        plsc.subcore_barrier()
        flush_accN(num0_out)
        zero_accN()
        plsc.subcore_barrier()
        do_pass(h1_hbm, False)
        plsc.subcore_barrier()
        flush_accN(num1_out)

        pltpu.sync_copy(accD.at[pl.ds(sid * RPT, RPT)], zden)
        pltpu.sync_copy(zden,
                        den_out.at[pl.ds(cid * NPAD + sid * RPT, RPT)])

    return pl.kernel(body, out_type=out_type, mesh=mesh,
                     scratch_types=scratch,
                     compiler_params=pltpu.CompilerParams(
                         needs_layout_passes=False,
                         use_tc_tiling_on_sc=False))


_get_sc_edge = functools.lru_cache(maxsize=None)(_make_sc_edge)


def kernel(x, edge_index, W1, a_src1, a_dst1, b1,
           W2, a_src2, a_dst2, b2, Wv, a_srcv, a_dstv, bv):
    src = edge_index[0].reshape(NW, NCHUNK, CH)
    dst = edge_index[1].reshape(NW, NCHUNK, CH)

    def den2d(den_flat):
        return den_flat.reshape(NC, NPAD)[:, :N].T

    h1a, h1b, av1, bv1 = _tc_pre(x, W1, a_src1.reshape(D, 1),
                                 a_dst1.reshape(D, 1))
    n1a, n1b, den1 = _get_sc_edge(0)(h1a, h1b, av1.reshape(N),
                                     bv1.reshape(N), src, dst)

    h2a, h2b, av2, bv2 = _tc_mid(n1a, n1b, den2d(den1), b1.reshape(1, D),
                                 W2, a_src2.reshape(D, 1),
                                 a_dst2.reshape(D, 1))
    n2a, n2b, den2, ep2 = _get_sc_edge(1)(h2a, h2b, av2.reshape(N),
                                          bv2.reshape(N), src, dst)

    mean, h3a, h3b, av3, bv3, rcp2 = _tc_mid3(
        n2a, n2b, den2d(den2), b2.reshape(1, D), Wv,
        a_srcv.reshape(D, 1), a_dstv.reshape(D, 1))
    n3a, n3b, den3, alpha, c3 = _get_sc_edge(2)(h3a, h3b, av3.reshape(N),
                                                bv3.reshape(N), src, dst,
                                                ep2, rcp2.reshape(N))

    var = _tc_fin(n3a, n3b, den2d(den3), bv.reshape(1, D), h3a, h3b,
                  av3, bv3, c3)
    return (mean, var, alpha.reshape(E))


# speedup vs baseline: 40.2930x; 1.0560x over previous
import functools

import jax
import jax.numpy as jnp
from jax import lax
from jax.experimental import pallas as pl
from jax.experimental.pallas import tpu as pltpu
from jax.experimental.pallas import tpu_sc as plsc

N = 10000
E = 320000
D = 128
DH = 64

NC = 2
NS = 16
L = 16
NW = NC * NS
EPT = E // NW
CH = 80
NCHUNK = EPT // CH
NB = 5
NG = NCHUNK // NB
NPAD = 10240
RPT = NPAD // NS
ZR = 128
BN = 1000
EPS = 1e-16
F32 = jnp.float32


def _lrelu(x):
    return jnp.maximum(x, 0.2 * x)


_B_X = pl.BlockSpec((BN, D), lambda i: (i, 0))
_B_W = pl.BlockSpec((D, D), lambda i: (0, 0))
_B_A = pl.BlockSpec((D, 1), lambda i: (0, 0))
_B_BIAS = pl.BlockSpec((1, D), lambda i: (0, 0))
_B_NUM = pl.BlockSpec((NC, BN, DH), lambda i: (0, i, 0))
_B_DEN = pl.BlockSpec((BN, NC), lambda i: (i, 0))
_B_H = pl.BlockSpec((BN, DH), lambda i: (i, 0))
_B_V = pl.BlockSpec((BN, 1), lambda i: (i, 0))
_B_C = pl.BlockSpec((1, L), lambda i: (0, 0))

_GRID = (N // BN,)

_PROJ_OUT = [
    jax.ShapeDtypeStruct((N, DH), F32),
    jax.ShapeDtypeStruct((N, DH), F32),
    jax.ShapeDtypeStruct((N, 1), F32),
    jax.ShapeDtypeStruct((N, 1), F32),
]
_PROJ_OUT_SPECS = [_B_H, _B_H, _B_V, _B_V]


def _proj_body(h, asr_ref, ads_ref, h0_ref, h1_ref, av_ref, bv_ref):
    h0_ref[...] = h[:, :DH]
    h1_ref[...] = h[:, DH:]
    av_ref[...] = jnp.dot(h, asr_ref[...], preferred_element_type=F32)
    bv_ref[...] = jnp.dot(h, ads_ref[...], preferred_element_type=F32)


def _tc_pre_body(x_ref, w_ref, asr_ref, ads_ref,
                 h0_ref, h1_ref, av_ref, bv_ref):
    h = jnp.dot(x_ref[...], w_ref[...], preferred_element_type=F32)
    _proj_body(h, asr_ref, ads_ref, h0_ref, h1_ref, av_ref, bv_ref)


def _tc_pre(x, w, asr, ads):
    return pl.pallas_call(
        _tc_pre_body, grid=_GRID,
        in_specs=[_B_X, _B_W, _B_A, _B_A],
        out_specs=_PROJ_OUT_SPECS,
        out_shape=_PROJ_OUT,
    )(x, w, asr, ads)


def _numsum(num0_ref, num1_ref):
    return jnp.concatenate(
        [num0_ref[0] + num0_ref[1], num1_ref[0] + num1_ref[1]], axis=1)


def _tc_mid_body(num0_ref, num1_ref, den_ref, b_ref, w_ref, asr_ref, ads_ref,
                 h0_ref, h1_ref, av_ref, bv_ref):
    ns = _numsum(num0_ref, num1_ref)
    dsum = den_ref[:, 0:1] + den_ref[:, 1:2] + EPS
    o = ns / dsum + b_ref[...]
    o = jnp.maximum(o, 0.0)
    h = jnp.dot(o, w_ref[...], preferred_element_type=F32)
    _proj_body(h, asr_ref, ads_ref, h0_ref, h1_ref, av_ref, bv_ref)


def _tc_mid(num0, num1, den, b, w, asr, ads):
    return pl.pallas_call(
        _tc_mid_body, grid=_GRID,
        in_specs=[_B_NUM, _B_NUM, _B_DEN, _B_BIAS, _B_W, _B_A, _B_A],
        out_specs=_PROJ_OUT_SPECS,
        out_shape=_PROJ_OUT,
    )(num0, num1, den, b, w, asr, ads)


def _tc_mid3_body(num0_ref, num1_ref, den_ref, b_ref, w_ref, asr_ref, ads_ref,
                  mean_ref, h0_ref, h1_ref, av_ref, bv_ref, rcp_ref):
    ns = _numsum(num0_ref, num1_ref)
    dsum = den_ref[:, 0:1] + den_ref[:, 1:2] + EPS
    rcp = 1.0 / dsum
    mean = ns * rcp + b_ref[...]
    mean_ref[...] = mean
    rcp_ref[...] = rcp
    h = jnp.dot(mean, w_ref[...], preferred_element_type=F32)
    _proj_body(h, asr_ref, ads_ref, h0_ref, h1_ref, av_ref, bv_ref)


def _tc_mid3(num0, num1, den, b, w, asr, ads):
    return pl.pallas_call(
        _tc_mid3_body, grid=_GRID,
        in_specs=[_B_NUM, _B_NUM, _B_DEN, _B_BIAS, _B_W, _B_A, _B_A],
        out_specs=[pl.BlockSpec((BN, D), lambda i: (i, 0))]
        + _PROJ_OUT_SPECS + [_B_V],
        out_shape=[jax.ShapeDtypeStruct((N, D), F32)] + _PROJ_OUT
        + [jax.ShapeDtypeStruct((N, 1), F32)],
    )(num0, num1, den, b, w, asr, ads)


def _tc_fin_body(num0_ref, num1_ref, den_ref, b_ref, h0_ref, h1_ref,
                 av_ref, bv_ref, c_ref, var_ref):
    s = av_ref[...] + bv_ref[...]
    es = jnp.exp(_lrelu(s) - c_ref[0:1, 0:1])
    h = jnp.concatenate([h0_ref[...], h1_ref[...]], axis=1)
    numt = _numsum(num0_ref, num1_ref) + es * h
    dent = den_ref[:, 0:1] + den_ref[:, 1:2] + es + EPS
    var_ref[...] = numt / dent + b_ref[...]


def _tc_fin(num0, num1, den, b, h0, h1, av, bv, c):
    return pl.pallas_call(
        _tc_fin_body, grid=_GRID,
        in_specs=[_B_NUM, _B_NUM, _B_DEN, _B_BIAS, _B_H, _B_H, _B_V, _B_V,
                  _B_C],
        out_specs=pl.BlockSpec((BN, D), lambda i: (i, 0)),
        out_shape=jax.ShapeDtypeStruct((N, D), F32),
    )(num0, num1, den, b, h0, h1, av, bv, c)


def _make_sc_edge(mode):
    mesh = plsc.VectorSubcoreMesh(core_axis_name="c", subcore_axis_name="s",
                                  num_cores=NC, num_subcores=NS)

    out_type = [
        jax.ShapeDtypeStruct((NC, NPAD, DH), F32),
        jax.ShapeDtypeStruct((NC, NPAD, DH), F32),
        jax.ShapeDtypeStruct((NC * NPAD,), F32),
    ]
    if mode == 1:
        out_type.append(jax.ShapeDtypeStruct((NW, NCHUNK, CH), F32))
    if mode == 2:
        out_type.append(jax.ShapeDtypeStruct((NW, NCHUNK, CH), F32))
        out_type.append(jax.ShapeDtypeStruct((1, L), F32))

    scratch = [
        pltpu.VMEM((NCHUNK, CH), jnp.int32),
        pltpu.VMEM((NCHUNK, CH), jnp.int32),
        pltpu.VMEM((N,), F32),
        pltpu.VMEM((N,), F32),
        pltpu.VMEM((NB, CH, DH), F32),
        pltpu.VMEM((NCHUNK, CH), F32),
        pltpu.VMEM((ZR, DH), F32),
        pltpu.VMEM((RPT,), F32),
        pltpu.VMEM((1, L), F32),
        pltpu.VMEM_SHARED((NPAD, DH), F32),
        pltpu.VMEM_SHARED((NPAD,), F32),
    ] + [pltpu.SemaphoreType.DMA] * (2 * NB + 1)

    def body(*refs):
        if mode == 0:
            (h0_hbm, h1_hbm, asrc_hbm, adst_hbm, src_hbm, dst_hbm,
             num0_out, num1_out, den_out,
             src_l, dst_l, asrc_l, adst_l, rows, epf, zrow, zden, cw_l,
             accN, accD, *sems) = refs
        elif mode == 1:
            (h0_hbm, h1_hbm, asrc_hbm, adst_hbm, src_hbm, dst_hbm,
             num0_out, num1_out, den_out, ep_out,
             src_l, dst_l, asrc_l, adst_l, rows, epf, zrow, zden, cw_l,
             accN, accD, *sems) = refs
        else:
            (h0_hbm, h1_hbm, asrc_hbm, adst_hbm, src_hbm, dst_hbm,
             ep2_hbm, rcp_hbm,
             num0_out, num1_out, den_out, alpha_out, c_out,
             src_l, dst_l, asrc_l, adst_l, rows, epf, zrow, zden, cw_l,
             accN, accD, *sems) = refs
        gsems = sems[:NB]
        ssems = sems[NB:2 * NB]
        dsem = sems[2 * NB]

        cid = lax.axis_index("c")
        sid = lax.axis_index("s")
        wid = cid * NS + sid

        zero = jnp.zeros((L,), F32)

        pltpu.async_copy(src_hbm.at[wid], src_l, gsems[0])
        pltpu.async_copy(dst_hbm.at[wid], dst_l, gsems[1])
        pltpu.async_copy(adst_hbm, adst_l, gsems[2])
        if mode == 2:
            pltpu.async_copy(rcp_hbm, asrc_l, gsems[3])
            pltpu.async_copy(ep2_hbm.at[wid], epf, ssems[0])
        else:
            pltpu.async_copy(asrc_hbm, asrc_l, gsems[3])

        def zr_body(i, _):
            zrow[i // (DH // L), pl.ds((i % (DH // L)) * L, L)] = zero
            return 0
        lax.fori_loop(0, ZR * (DH // L), zr_body, 0)

        def zd_body(i, _):
            zden[pl.ds(i * L, L)] = zero
            return 0
        lax.fori_loop(0, RPT // L, zd_body, 0)

        def zero_accN():
            for k in range(RPT // ZR):
                pltpu.sync_copy(zrow, accN.at[pl.ds(sid * RPT + k * ZR, ZR)])

        def flush_accN(out):
            sl = pl.ds(sid * RPT, RPT)
            pltpu.sync_copy(accN.at[sl], out.at[cid, sl])

        zero_accN()
        pltpu.sync_copy(zden, accD.at[pl.ds(sid * RPT, RPT)])

        pltpu.make_async_copy(src_hbm.at[wid], src_l, gsems[0]).wait()
        pltpu.make_async_copy(dst_hbm.at[wid], dst_l, gsems[1]).wait()
        pltpu.make_async_copy(adst_hbm, adst_l, gsems[2]).wait()
        if mode == 2:
            pltpu.make_async_copy(rcp_hbm, asrc_l, gsems[3]).wait()
            pltpu.make_async_copy(ep2_hbm.at[wid], epf, ssems[0]).wait()
        else:
            pltpu.make_async_copy(asrc_hbm, asrc_l, gsems[3]).wait()

        if mode == 2:
            def apass(j, _):
                for g in range(CH // L):
                    dv = dst_l[j, pl.ds(g * L, L)]
                    rv = plsc.load_gather(asrc_l, [dv])
                    epf[j, pl.ds(g * L, L)] = epf[j, pl.ds(g * L, L)] * rv
                return 0
            lax.fori_loop(0, NCHUNK, apass, 0)
            pltpu.sync_copy(epf, alpha_out.at[wid])
            pltpu.sync_copy(asrc_hbm, asrc_l)

        plsc.subcore_barrier()

        ninf = jnp.full((L,), -3.0e38, F32)

        def mx_body(i, acc):
            ma, mb = acc
            return (jnp.maximum(ma, asrc_l[pl.ds(i * L, L)]),
                    jnp.maximum(mb, adst_l[pl.ds(i * L, L)]))
        ma, mb = lax.fori_loop(0, N // L, mx_body, (ninf, ninf))
        cs = jnp.max(ma) + jnp.max(mb)
        cval = jnp.full((L,), _lrelu(cs), F32)

        if mode == 2:
            cw_l[0, :] = cval

            @pl.when(wid == 0)
            def _():
                pltpu.sync_copy(cw_l, c_out)

        def do_pass(h_hbm, first):
            def fire_gather(j, b):
                pltpu.async_copy(h_hbm.at[src_l.at[j]], rows.at[b],
                                 gsems[b])

            def wait_gather(j, b):
                pltpu.make_async_copy(h_hbm.at[src_l.at[j]], rows.at[b],
                                      gsems[b]).wait()

            def fire_scat(j, b):
                pltpu.async_copy(rows.at[b], accN.at[dst_l.at[j]],
                                 ssems[b], add=True)

            def wait_scat(j, b):
                pltpu.make_async_copy(rows.at[b], accN.at[dst_l.at[j]],
                                      ssems[b]).wait()

            def wait_den(j):
                pltpu.make_async_copy(epf.at[j], accD.at[dst_l.at[j]],
                                      dsem).wait()

            def step(j, b, wait_prev_den):
                wait_gather(j, b)
                if first:
                    for g in range(CH // L):
                        sv = src_l[j, pl.ds(g * L, L)]
                        dv = dst_l[j, pl.ds(g * L, L)]
                        s = (plsc.load_gather(asrc_l, [sv])
                             + plsc.load_gather(adst_l, [dv]))
                        epf[j, pl.ds(g * L, L)] = jnp.exp(_lrelu(s) - cval)
                    if wait_prev_den:
                        wait_den(j - 1)
                    pltpu.async_copy(epf.at[j], accD.at[dst_l.at[j]],
                                     dsem, add=True)

                wj = jnp.full((L,), j, jnp.int32)

                def scale(r4, _):
                    r = r4 * 4
                    for u in range(4):
                        w = plsc.load_gather(
                            epf, [wj, jnp.full((L,), r + u, jnp.int32)])
                        for q in range(DH // L):
                            rows[b, r + u, pl.ds(q * L, L)] = (
                                rows[b, r + u, pl.ds(q * L, L)] * w)
                    return 0
                lax.fori_loop(0, CH // 4, scale, 0)
                fire_scat(j, b)

            for b in range(NB - 1):
                fire_gather(b, b)
            for b in range(NB):
                step(b, b, wait_prev_den=(first and b > 0))
                if b > 0:
                    wait_scat(b - 1, b - 1)
                fire_gather(b + NB - 1, (b - 1) % NB)

            def group(gidx, _):
                j0 = gidx * NB
                for b in range(NB):
                    j = j0 + b
                    step(j, b, wait_prev_den=first)
                    wait_scat(j - 1, (b + NB - 1) % NB)
                    fire_gather(j + NB - 1, (b + NB - 1) % NB)
                return 0
            lax.fori_loop(1, NG - 1, group, 0)

            j0 = NCHUNK - NB
            step(j0, 0, wait_prev_den=first)
            wait_scat(j0 - 1, NB - 1)
            fire_gather(NCHUNK - 1, NB - 1)
            for b in range(1, NB):
                step(j0 + b, b, wait_prev_den=first)
            for b in range(NB):
                wait_scat(j0 + b, b)
            if first:
                wait_den(NCHUNK - 1)

        do_pass(h0_hbm, True)
        if mode == 1:
            pltpu.sync_copy(epf, ep_out.at[wid])
        plsc.subcore_barrier()
        flush_accN(num0_out)
        zero_accN()
        plsc.subcore_barrier()
        do_pass(h1_hbm, False)
        plsc.subcore_barrier()
        flush_accN(num1_out)

        pltpu.sync_copy(accD.at[pl.ds(sid * RPT, RPT)],
                        den_out.at[pl.ds(cid * NPAD + sid * RPT, RPT)])

    return pl.kernel(body, out_type=out_type, mesh=mesh,
                     scratch_types=scratch,
                     compiler_params=pltpu.CompilerParams(
                         needs_layout_passes=False,
                         use_tc_tiling_on_sc=False))


_get_sc_edge = functools.lru_cache(maxsize=None)(_make_sc_edge)


def kernel(x, edge_index, W1, a_src1, a_dst1, b1,
           W2, a_src2, a_dst2, b2, Wv, a_srcv, a_dstv, bv):
    src = edge_index[0].reshape(NW, NCHUNK, CH)
    dst = edge_index[1].reshape(NW, NCHUNK, CH)

    def den2d(den_flat):
        return den_flat.reshape(NC, NPAD)[:, :N].T

    h1a, h1b, av1, bv1 = _tc_pre(x, W1, a_src1.reshape(D, 1),
                                 a_dst1.reshape(D, 1))
    n1a, n1b, den1 = _get_sc_edge(0)(h1a, h1b, av1.reshape(N),
                                     bv1.reshape(N), src, dst)

    h2a, h2b, av2, bv2 = _tc_mid(n1a, n1b, den2d(den1), b1.reshape(1, D),
                                 W2, a_src2.reshape(D, 1),
                                 a_dst2.reshape(D, 1))
    n2a, n2b, den2, ep2 = _get_sc_edge(1)(h2a, h2b, av2.reshape(N),
                                          bv2.reshape(N), src, dst)

    mean, h3a, h3b, av3, bv3, rcp2 = _tc_mid3(
        n2a, n2b, den2d(den2), b2.reshape(1, D), Wv,
        a_srcv.reshape(D, 1), a_dstv.reshape(D, 1))
    n3a, n3b, den3, alpha, c3 = _get_sc_edge(2)(h3a, h3b, av3.reshape(N),
                                                bv3.reshape(N), src, dst,
                                                ep2, rcp2.reshape(N))

    var = _tc_fin(n3a, n3b, den2d(den3), bv.reshape(1, D), h3a, h3b,
                  av3, bv3, c3)
    return (mean, var, alpha.reshape(E))
